# Initial kernel scaffold; baseline (speedup 1.0000x reference)
import functools

import jax
import jax.numpy as jnp
from jax import lax
from jax.experimental import pallas as pl
from jax.experimental.pallas import tpu as pltpu
from jax.experimental.pallas import tpu_sc as plsc

N = 10000
E = 160000
F = 256
H = 512

NSUB = 16
NCHUNK = 125
CW = 80
NPAD = 10240
RPT = NPAD // NSUB
BN = 400
GRID = N // BN

_mesh = plsc.VectorSubcoreMesh(core_axis_name="c", subcore_axis_name="s")


def _zero_vec(ref, nwords):
    def body(k, carry):
        ref[pl.ds(k * 16, 16)] = jnp.zeros((16,), jnp.float32)
        return carry
    lax.fori_loop(0, nwords // 16, body, 0)


@functools.partial(
    pl.kernel,
    out_type=jax.ShapeDtypeStruct((2, NPAD), jnp.float32),
    mesh=_mesh,
    scratch_types=[
        pltpu.VMEM((NCHUNK, CW), jnp.int32),
        pltpu.VMEM((CW,), jnp.float32),
        pltpu.VMEM((RPT,), jnp.float32),
        pltpu.VMEM_SHARED((NPAD,), jnp.float32),
    ],
)
def _degrees(e4, deg2, idx_v, ones_v, buf, acc):
    c = lax.axis_index("c")
    s = lax.axis_index("s")
    pltpu.sync_copy(e4.at[c, s], idx_v)
    for l in range(CW // 16):
        ones_v[pl.ds(l * 16, 16)] = jnp.ones((16,), jnp.float32)
    _zero_vec(buf, RPT)
    pltpu.sync_copy(buf, acc.at[pl.ds(s * RPT, RPT)])
    plsc.subcore_barrier()

    def body(j, carry):
        pltpu.sync_copy(ones_v, acc.at[idx_v.at[j]], add=True)
        return carry
    lax.fori_loop(0, NCHUNK, body, 0)
    plsc.subcore_barrier()
    pltpu.sync_copy(acc.at[pl.ds(s * RPT, RPT)], buf)
    pltpu.sync_copy(buf, deg2.at[c, pl.ds(s * RPT, RPT)])


def _normalize_body(feat, dego, degi, y, nsrc, ndst):
    ns = lax.rsqrt(jnp.maximum(dego[...], 1.0))
    nd = lax.rsqrt(jnp.maximum(degi[...], 1.0))
    y[0] = feat[...] * ns
    nsrc[...] = ns
    ndst[...] = nd


def _normalize(features, deg_out, deg_in):
    return pl.pallas_call(
        _normalize_body,
        grid=(GRID, 2),
        in_specs=[
            pl.BlockSpec((BN, 128), lambda i, h: (i, h)),
            pl.BlockSpec((BN, 1), lambda i, h: (i, 0)),
            pl.BlockSpec((BN, 1), lambda i, h: (i, 0)),
        ],
        out_specs=[
            pl.BlockSpec((1, BN, 128), lambda i, h: (h, i, 0)),
            pl.BlockSpec((BN, 1), lambda i, h: (i, 0)),
            pl.BlockSpec((BN, 1), lambda i, h: (i, 0)),
        ],
        out_shape=[
            jax.ShapeDtypeStruct((2, N, 128), jnp.float32),
            jax.ShapeDtypeStruct((N, 1), jnp.float32),
            jax.ShapeDtypeStruct((N, 1), jnp.float32),
        ],
    )(features, deg_out, deg_in)


KSUP = 5

@functools.partial(
    pl.kernel,
    out_type=jax.ShapeDtypeStruct((2, NPAD, 128), jnp.float32),
    mesh=_mesh,
    scratch_types=[
        pltpu.VMEM((NCHUNK, CW), jnp.int32),
        pltpu.VMEM((NCHUNK, CW), jnp.int32),
        pltpu.VMEM((KSUP, CW, 128), jnp.float32),
        pltpu.VMEM((16, 128), jnp.float32),
        pltpu.VMEM_SHARED((NPAD, 128), jnp.float32),
        pltpu.SemaphoreType.DMA,
    ],
)
def _aggregate(yc, s4, d3, agg, src_v, dst_v, st, zb, acc, gsem):
    c = lax.axis_index("c")
    s = lax.axis_index("s")
    pltpu.sync_copy(s4.at[c, s], src_v)
    pltpu.sync_copy(d3.at[s], dst_v)

    def zrow(r, carry):
        for l in range(8):
            zb[r, pl.ds(l * 16, 16)] = jnp.zeros((16,), jnp.float32)
        return carry
    lax.fori_loop(0, 16, zrow, 0)

    def zcp(t, carry):
        pltpu.sync_copy(zb, acc.at[pl.ds(s * RPT + t * 16, 16)])
        return carry
    lax.fori_loop(0, RPT // 16, zcp, 0)
    plsc.subcore_barrier()

    def super_step(t, carry):
        base = t * KSUP
        ds = [
            pltpu.async_copy(yc.at[src_v.at[base + b]], st.at[b], gsem)
            for b in range(KSUP)
        ]
        for d in ds:
            d.wait()
        for b in range(KSUP):
            pltpu.sync_copy(st.at[b], acc.at[dst_v.at[base + b]], add=True)
        return carry
    lax.fori_loop(0, NCHUNK // KSUP, super_step, 0)
    plsc.subcore_barrier()

    def out_cp(t, carry):
        pltpu.sync_copy(acc.at[pl.ds(s * RPT + t * 16, 16)], zb)
        pltpu.sync_copy(zb, agg.at[c, pl.ds(s * RPT + t * 16, 16)])
        return carry
    lax.fori_loop(0, RPT // 16, out_cp, 0)


def _dense_body(a0, a1, nd, ns, w1, b1, w2, g):
    h = lax.dot_general(
        a0[...], w1[pl.ds(0, 128), :], (((1,), (0,)), ((), ())),
        precision=lax.Precision.HIGHEST, preferred_element_type=jnp.float32)
    h = h + lax.dot_general(
        a1[...], w1[pl.ds(128, 128), :], (((1,), (0,)), ((), ())),
        precision=lax.Precision.HIGHEST, preferred_element_type=jnp.float32)
    h = h * nd[...] + b1[...]
    h = jnp.maximum(h, 0.0) * ns[...]
    g[...] = lax.dot_general(
        h, w2[...], (((1,), (0,)), ((), ())),
        precision=lax.Precision.HIGHEST, preferred_element_type=jnp.float32)


def _dense(a0, a1, ndst, nsrc, W1, b1, W2):
    return pl.pallas_call(
        _dense_body,
        grid=(GRID,),
        in_specs=[
            pl.BlockSpec((BN, 128), lambda i: (i, 0)),
            pl.BlockSpec((BN, 128), lambda i: (i, 0)),
            pl.BlockSpec((BN, 1), lambda i: (i, 0)),
            pl.BlockSpec((BN, 1), lambda i: (i, 0)),
            pl.BlockSpec((F, H), lambda i: (0, 0)),
            pl.BlockSpec((1, H), lambda i: (0, 0)),
            pl.BlockSpec((H, 1), lambda i: (0, 0)),
        ],
        out_specs=pl.BlockSpec((BN, 1), lambda i: (i, 0)),
        out_shape=jax.ShapeDtypeStruct((N, 1), jnp.float32),
    )(a0, a1, ndst, nsrc, W1, b1, W2)


@functools.partial(
    pl.kernel,
    out_type=jax.ShapeDtypeStruct((2, NPAD), jnp.float32),
    mesh=_mesh,
    scratch_types=[
        pltpu.VMEM((NCHUNK, CW), jnp.int32),
        pltpu.VMEM((NCHUNK, CW), jnp.int32),
        pltpu.VMEM((CW,), jnp.float32),
        pltpu.VMEM((RPT,), jnp.float32),
        pltpu.VMEM((RPT,), jnp.float32),
        pltpu.VMEM((RPT,), jnp.float32),
        pltpu.VMEM((16,), jnp.float32),
        pltpu.VMEM_SHARED((NPAD,), jnp.float32),
        pltpu.SemaphoreType.DMA,
    ],
)
def _layer2(g1, s3, d3, ndp, b2h, o2, src_v, dst_v, gbuf,
            lbuf, nbuf, obuf, b2v, acc, gsem):
    c = lax.axis_index("c")
    s = lax.axis_index("s")
    pltpu.sync_copy(s3.at[s], src_v)
    pltpu.sync_copy(d3.at[s], dst_v)
    pltpu.sync_copy(b2h, b2v)
    _zero_vec(obuf, RPT)
    pltpu.sync_copy(obuf, acc.at[pl.ds(s * RPT, RPT)])
    plsc.subcore_barrier()

    def body(j, carry):
        pltpu.async_copy(g1.at[src_v.at[j]], gbuf, gsem).wait()
        pltpu.sync_copy(gbuf, acc.at[dst_v.at[j]], add=True)
        return carry
    lax.fori_loop(0, NCHUNK, body, 0)
    plsc.subcore_barrier()

    pltpu.sync_copy(acc.at[pl.ds(s * RPT, RPT)], lbuf)
    pltpu.sync_copy(ndp.at[pl.ds(s * RPT, RPT)], nbuf)
    b2r = b2v[...]

    def scale(k, carry):
        sl = pl.ds(k * 16, 16)
        obuf[sl] = lbuf[sl] * nbuf[sl] + b2r
        return carry
    lax.fori_loop(0, RPT // 16, scale, 0)
    pltpu.sync_copy(obuf, o2.at[c, pl.ds(s * RPT, RPT)])


def kernel(features, edge_index, W1, b1, W2, b2):
    src = edge_index[0].astype(jnp.int32)
    dst = edge_index[1].astype(jnp.int32)
    src3 = src.reshape(NSUB, NCHUNK, CW)
    dst3 = dst.reshape(NSUB, NCHUNK, CW)
    e4 = jnp.stack([src3, dst3])
    s4 = jnp.stack([src3, src3 + N])

    deg2 = _degrees(e4)
    deg_out = deg2[0, :N, None]
    deg_in = deg2[1, :N, None]

    ycat, nsrc, ndst = _normalize(features, deg_out, deg_in)
    yflat = ycat.reshape(2 * N, 128)

    agg = _aggregate(yflat, s4, dst3)
    g = _dense(agg[0, :N], agg[1, :N], ndst, nsrc, W1, b1.reshape(1, H), W2)

    ndp = jnp.concatenate([ndst[:, 0], jnp.zeros((NPAD - N,), jnp.float32)])
    b2h = jnp.broadcast_to(b2, (16,))
    o2 = _layer2(g[:, 0], src3, dst3, ndp, b2h)
    return o2[0, :N].reshape(N, 1)


# speedup vs baseline: 6.8799x; 6.8799x over previous
import functools

import jax
import jax.numpy as jnp
from jax import lax
from jax.experimental import pallas as pl
from jax.experimental.pallas import tpu as pltpu
from jax.experimental.pallas import tpu_sc as plsc

N = 10000
E = 160000
F = 256
H = 512

NSUB = 16
NCHUNK = 125
CW = 80
NPAD = 10240
RPT = NPAD // NSUB
BN = 400
GRID = N // BN

_mesh = plsc.VectorSubcoreMesh(core_axis_name="c", subcore_axis_name="s")


def _zero_vec(ref, nwords):
    def body(k, carry):
        ref[pl.ds(k * 16, 16)] = jnp.zeros((16,), jnp.float32)
        return carry
    lax.fori_loop(0, nwords // 16, body, 0)


@functools.partial(
    pl.kernel,
    out_type=jax.ShapeDtypeStruct((2, NPAD), jnp.float32),
    mesh=_mesh,
    scratch_types=[
        pltpu.VMEM((NCHUNK, CW), jnp.int32),
        pltpu.VMEM((CW,), jnp.float32),
        pltpu.VMEM((RPT,), jnp.float32),
        pltpu.VMEM_SHARED((NPAD,), jnp.float32),
    ],
)
def _degrees(e4, deg2, idx_v, ones_v, buf, acc):
    c = lax.axis_index("c")
    s = lax.axis_index("s")
    pltpu.sync_copy(e4.at[c, s], idx_v)
    for l in range(CW // 16):
        ones_v[pl.ds(l * 16, 16)] = jnp.ones((16,), jnp.float32)
    _zero_vec(buf, RPT)
    pltpu.sync_copy(buf, acc.at[pl.ds(s * RPT, RPT)])
    plsc.subcore_barrier()

    def body(j, carry):
        pltpu.sync_copy(ones_v, acc.at[idx_v.at[j]], add=True)
        return carry
    lax.fori_loop(0, NCHUNK, body, 0)
    plsc.subcore_barrier()
    pltpu.sync_copy(acc.at[pl.ds(s * RPT, RPT)], buf)
    pltpu.sync_copy(buf, deg2.at[c, pl.ds(s * RPT, RPT)])


QW = 64
NQ = F // QW


def _normalize_body(feat, dego, degi, y, nsrc, ndst):
    ns = lax.rsqrt(jnp.maximum(dego[...], 1.0))
    nd = lax.rsqrt(jnp.maximum(degi[...], 1.0))
    ysc = feat[...] * ns
    y[0] = ysc[:, :QW]
    y[1] = ysc[:, QW:]
    nsrc[...] = ns
    ndst[...] = nd


def _normalize(features, deg_out, deg_in):
    return pl.pallas_call(
        _normalize_body,
        grid=(GRID, 2),
        in_specs=[
            pl.BlockSpec((BN, 128), lambda i, h: (i, h)),
            pl.BlockSpec((BN, 1), lambda i, h: (i, 0)),
            pl.BlockSpec((BN, 1), lambda i, h: (i, 0)),
        ],
        out_specs=[
            pl.BlockSpec((2, BN, QW), lambda i, h: (h, i, 0)),
            pl.BlockSpec((BN, 1), lambda i, h: (i, 0)),
            pl.BlockSpec((BN, 1), lambda i, h: (i, 0)),
        ],
        out_shape=[
            jax.ShapeDtypeStruct((NQ, N, QW), jnp.float32),
            jax.ShapeDtypeStruct((N, 1), jnp.float32),
            jax.ShapeDtypeStruct((N, 1), jnp.float32),
        ],
    )(features, deg_out, deg_in)


KSUP = 5

@functools.partial(
    pl.kernel,
    out_type=jax.ShapeDtypeStruct((NQ, NPAD, QW), jnp.float32),
    mesh=_mesh,
    scratch_types=[
        pltpu.VMEM((NCHUNK, CW), jnp.int32),
        pltpu.VMEM((NCHUNK, CW), jnp.int32),
        pltpu.VMEM((KSUP, CW, QW), jnp.float32),
        pltpu.VMEM((16, QW), jnp.float32),
        pltpu.VMEM_SHARED((NPAD, QW), jnp.float32),
        pltpu.SemaphoreType.DMA,
    ],
    compiler_params=pltpu.CompilerParams(use_tc_tiling_on_sc=False),
)
def _aggregate(yc, s8, d3, agg, src_v, dst_v, st, zb, acc, gsem):
    c = lax.axis_index("c")
    s = lax.axis_index("s")
    pltpu.sync_copy(d3.at[s], dst_v)

    def zrow(r, carry):
        for l in range(QW // 16):
            zb[r, pl.ds(l * 16, 16)] = jnp.zeros((16,), jnp.float32)
        return carry
    lax.fori_loop(0, 16, zrow, 0)

    for p in range(2):
        q = 2 * c + p
        pltpu.sync_copy(s8.at[q, s], src_v)

        def zcp(t, carry):
            pltpu.sync_copy(zb, acc.at[pl.ds(s * RPT + t * 16, 16)])
            return carry
        lax.fori_loop(0, RPT // 16, zcp, 0)
        plsc.subcore_barrier()

        def super_step(t, carry):
            base = t * KSUP
            ds = [
                pltpu.async_copy(yc.at[src_v.at[base + b]], st.at[b], gsem)
                for b in range(KSUP)
            ]
            for d in ds:
                d.wait()
            for b in range(KSUP):
                pltpu.sync_copy(st.at[b], acc.at[dst_v.at[base + b]], add=True)
            return carry
        lax.fori_loop(0, NCHUNK // KSUP, super_step, 0)
        plsc.subcore_barrier()

        def out_cp(t, carry):
            pltpu.sync_copy(acc.at[pl.ds(s * RPT + t * 16, 16)], zb)
            pltpu.sync_copy(zb, agg.at[q, pl.ds(s * RPT + t * 16, 16)])
            return carry
        lax.fori_loop(0, RPT // 16, out_cp, 0)
        if p == 0:
            lax.fori_loop(0, 16, zrow, 0)
            plsc.subcore_barrier()


def _dense_body(a0, a1, a2, a3, nd, ns, w1, b1, w2, g):
    a = jnp.concatenate([a0[...], a1[...], a2[...], a3[...]], axis=1)
    h = lax.dot_general(
        a, w1[...], (((1,), (0,)), ((), ())),
        precision=lax.Precision.HIGHEST, preferred_element_type=jnp.float32)
    h = h * nd[...] + b1[...]
    h = jnp.maximum(h, 0.0) * ns[...]
    g[...] = lax.dot_general(
        h, w2[...], (((1,), (0,)), ((), ())),
        precision=lax.Precision.HIGHEST, preferred_element_type=jnp.float32)


def _dense(aq, ndst, nsrc, W1, b1, W2):
    return pl.pallas_call(
        _dense_body,
        grid=(GRID,),
        in_specs=[
            pl.BlockSpec((BN, QW), lambda i: (i, 0)),
            pl.BlockSpec((BN, QW), lambda i: (i, 0)),
            pl.BlockSpec((BN, QW), lambda i: (i, 0)),
            pl.BlockSpec((BN, QW), lambda i: (i, 0)),
            pl.BlockSpec((BN, 1), lambda i: (i, 0)),
            pl.BlockSpec((BN, 1), lambda i: (i, 0)),
            pl.BlockSpec((F, H), lambda i: (0, 0)),
            pl.BlockSpec((1, H), lambda i: (0, 0)),
            pl.BlockSpec((H, 1), lambda i: (0, 0)),
        ],
        out_specs=pl.BlockSpec((BN, 1), lambda i: (i, 0)),
        out_shape=jax.ShapeDtypeStruct((N, 1), jnp.float32),
    )(aq[0], aq[1], aq[2], aq[3], ndst, nsrc, W1, b1, W2)


@functools.partial(
    pl.kernel,
    out_type=jax.ShapeDtypeStruct((2, NPAD), jnp.float32),
    mesh=_mesh,
    scratch_types=[
        pltpu.VMEM((NCHUNK, CW), jnp.int32),
        pltpu.VMEM((NCHUNK, CW), jnp.int32),
        pltpu.VMEM((CW,), jnp.float32),
        pltpu.VMEM((RPT,), jnp.float32),
        pltpu.VMEM((RPT,), jnp.float32),
        pltpu.VMEM((RPT,), jnp.float32),
        pltpu.VMEM((16,), jnp.float32),
        pltpu.VMEM_SHARED((NPAD,), jnp.float32),
        pltpu.SemaphoreType.DMA,
    ],
)
def _layer2(g1, s3, d3, ndp, b2h, o2, src_v, dst_v, gbuf,
            lbuf, nbuf, obuf, b2v, acc, gsem):
    c = lax.axis_index("c")
    s = lax.axis_index("s")
    pltpu.sync_copy(s3.at[s], src_v)
    pltpu.sync_copy(d3.at[s], dst_v)
    pltpu.sync_copy(b2h, b2v)
    _zero_vec(obuf, RPT)
    pltpu.sync_copy(obuf, acc.at[pl.ds(s * RPT, RPT)])
    plsc.subcore_barrier()

    def body(j, carry):
        pltpu.async_copy(g1.at[src_v.at[j]], gbuf, gsem).wait()
        pltpu.sync_copy(gbuf, acc.at[dst_v.at[j]], add=True)
        return carry
    lax.fori_loop(0, NCHUNK, body, 0)
    plsc.subcore_barrier()

    pltpu.sync_copy(acc.at[pl.ds(s * RPT, RPT)], lbuf)
    pltpu.sync_copy(ndp.at[pl.ds(s * RPT, RPT)], nbuf)
    b2r = b2v[...]

    def scale(k, carry):
        sl = pl.ds(k * 16, 16)
        obuf[sl] = lbuf[sl] * nbuf[sl] + b2r
        return carry
    lax.fori_loop(0, RPT // 16, scale, 0)
    pltpu.sync_copy(obuf, o2.at[c, pl.ds(s * RPT, RPT)])


def kernel(features, edge_index, W1, b1, W2, b2):
    src = edge_index[0].astype(jnp.int32)
    dst = edge_index[1].astype(jnp.int32)
    src3 = src.reshape(NSUB, NCHUNK, CW)
    dst3 = dst.reshape(NSUB, NCHUNK, CW)
    e4 = jnp.stack([src3, dst3])
    s8 = jnp.stack([src3 + q * N for q in range(NQ)])

    deg2 = _degrees(e4)
    deg_out = deg2[0, :N, None]
    deg_in = deg2[1, :N, None]

    ycat, nsrc, ndst = _normalize(features, deg_out, deg_in)
    yflat = ycat.reshape(NQ * N, QW)

    agg = _aggregate(yflat, s8, dst3)
    g = _dense([agg[q, :N] for q in range(NQ)], ndst, nsrc, W1,
               b1.reshape(1, H), W2)

    ndp = jnp.concatenate([ndst[:, 0], jnp.zeros((NPAD - N,), jnp.float32)])
    b2h = jnp.broadcast_to(b2, (16,))
    o2 = _layer2(g[:, 0], src3, dst3, ndp, b2h)
    return o2[0, :N].reshape(N, 1)


# speedup vs baseline: 9.1774x; 1.3339x over previous
import functools

import jax
import jax.numpy as jnp
from jax import lax
from jax.experimental import pallas as pl
from jax.experimental.pallas import tpu as pltpu
from jax.experimental.pallas import tpu_sc as plsc

N = 10000
E = 160000
F = 256
H = 512

NSUB = 16
NCHUNK = 125
CW = 80
NPAD = 10240
RPT = NPAD // NSUB
BN = 400
GRID = N // BN

_mesh = plsc.VectorSubcoreMesh(core_axis_name="c", subcore_axis_name="s")


def _zero_vec(ref, nwords):
    def body(k, carry):
        ref[pl.ds(k * 16, 16)] = jnp.zeros((16,), jnp.float32)
        return carry
    lax.fori_loop(0, nwords // 16, body, 0)


@functools.partial(
    pl.kernel,
    out_type=jax.ShapeDtypeStruct((2, NPAD), jnp.float32),
    mesh=_mesh,
    scratch_types=[
        pltpu.VMEM((NCHUNK, CW), jnp.int32),
        pltpu.VMEM((CW,), jnp.float32),
        pltpu.VMEM((RPT,), jnp.float32),
        pltpu.VMEM_SHARED((NPAD,), jnp.float32),
    ],
)
def _degrees(e4, deg2, idx_v, ones_v, buf, acc):
    c = lax.axis_index("c")
    s = lax.axis_index("s")
    pltpu.sync_copy(e4.at[c, s], idx_v)
    for l in range(CW // 16):
        ones_v[pl.ds(l * 16, 16)] = jnp.ones((16,), jnp.float32)
    _zero_vec(buf, RPT)
    pltpu.sync_copy(buf, acc.at[pl.ds(s * RPT, RPT)])
    plsc.subcore_barrier()

    def body(j, carry):
        pltpu.sync_copy(ones_v, acc.at[idx_v.at[j]], add=True)
        return carry
    lax.fori_loop(0, NCHUNK, body, 0)
    plsc.subcore_barrier()
    pltpu.sync_copy(acc.at[pl.ds(s * RPT, RPT)], buf)
    pltpu.sync_copy(buf, deg2.at[c, pl.ds(s * RPT, RPT)])


QW = 64
NQ = F // QW


def _normalize_body(feat, dego, degi, y, nsrc, ndst):
    ns = lax.rsqrt(jnp.maximum(dego[...], 1.0))
    nd = lax.rsqrt(jnp.maximum(degi[...], 1.0))
    ysc = feat[...] * ns
    y[0] = ysc[:, :QW]
    y[1] = ysc[:, QW:]
    nsrc[...] = ns
    ndst[...] = nd


def _normalize(features, deg_out, deg_in):
    return pl.pallas_call(
        _normalize_body,
        grid=(GRID, 2),
        in_specs=[
            pl.BlockSpec((BN, 128), lambda i, h: (i, h)),
            pl.BlockSpec((BN, 1), lambda i, h: (i, 0)),
            pl.BlockSpec((BN, 1), lambda i, h: (i, 0)),
        ],
        out_specs=[
            pl.BlockSpec((2, BN, QW), lambda i, h: (h, i, 0)),
            pl.BlockSpec((BN, 1), lambda i, h: (i, 0)),
            pl.BlockSpec((BN, 1), lambda i, h: (i, 0)),
        ],
        out_shape=[
            jax.ShapeDtypeStruct((NQ, N, QW), jnp.float32),
            jax.ShapeDtypeStruct((N, 1), jnp.float32),
            jax.ShapeDtypeStruct((N, 1), jnp.float32),
        ],
    )(features, deg_out, deg_in)


KSUP = 5

NSUP = NCHUNK // KSUP

@functools.partial(
    pl.kernel,
    out_type=jax.ShapeDtypeStruct((NQ, NPAD, QW), jnp.float32),
    mesh=_mesh,
    scratch_types=[
        pltpu.VMEM((NCHUNK, CW), jnp.int32),
        pltpu.VMEM((NCHUNK, CW), jnp.int32),
        pltpu.VMEM((2, KSUP, CW, QW), jnp.float32),
        pltpu.VMEM((16, QW), jnp.float32),
        pltpu.VMEM_SHARED((NPAD, QW), jnp.float32),
        pltpu.SemaphoreType.DMA,
    ],
    compiler_params=pltpu.CompilerParams(use_tc_tiling_on_sc=False),
)
def _aggregate(yc, s8, d3, agg, src_v, dst_v, st, zb, acc, gsem):
    c = lax.axis_index("c")
    s = lax.axis_index("s")
    pltpu.sync_copy(d3.at[s], dst_v)

    def zrow(r, carry):
        for l in range(QW // 16):
            zb[r, pl.ds(l * 16, 16)] = jnp.zeros((16,), jnp.float32)
        return carry
    lax.fori_loop(0, 16, zrow, 0)

    def fire(t, buf):
        base = t * KSUP
        for b in range(KSUP):
            pltpu.async_copy(
                yc.at[src_v.at[base + b]], st.at[buf, b], gsem)

    def drain(buf):
        for b in range(KSUP):
            pltpu.make_async_copy(
                yc.at[pl.ds(0, CW)], st.at[buf, b], gsem).wait()

    for p in range(2):
        q = 2 * c + p
        pltpu.sync_copy(s8.at[q, s], src_v)

        def zcp(t, carry):
            pltpu.sync_copy(zb, acc.at[pl.ds(s * RPT + t * 16, 16)])
            return carry
        lax.fori_loop(0, RPT // 16, zcp, 0)
        plsc.subcore_barrier()

        fire(0, 0)

        def super_step(t, carry):
            pp = lax.rem(t, 2)
            drain(pp)

            @pl.when(t + 1 < NSUP)
            def _():
                fire(t + 1, 1 - pp)
            base = t * KSUP
            for b in range(KSUP):
                pltpu.sync_copy(
                    st.at[pp, b], acc.at[dst_v.at[base + b]], add=True)
            return carry
        lax.fori_loop(0, NSUP, super_step, 0)
        plsc.subcore_barrier()

        def out_cp(t, carry):
            pltpu.sync_copy(acc.at[pl.ds(s * RPT + t * 16, 16)], zb)
            pltpu.sync_copy(zb, agg.at[q, pl.ds(s * RPT + t * 16, 16)])
            return carry
        lax.fori_loop(0, RPT // 16, out_cp, 0)
        if p == 0:
            lax.fori_loop(0, 16, zrow, 0)
            plsc.subcore_barrier()


def _dense_body(a0, a1, a2, a3, nd, ns, w1, b1, w2, g):
    a = jnp.concatenate([a0[0], a1[0], a2[0], a3[0]], axis=1)
    h = lax.dot_general(
        a, w1[...], (((1,), (0,)), ((), ())),
        precision=lax.Precision.HIGHEST, preferred_element_type=jnp.float32)
    h = h * nd[...] + b1[...]
    h = jnp.maximum(h, 0.0) * ns[...]
    g[...] = lax.dot_general(
        h, w2[...], (((1,), (0,)), ((), ())),
        precision=lax.Precision.HIGHEST, preferred_element_type=jnp.float32)


def _dense(agg, ndst, nsrc, W1, b1, W2):
    qspecs = [
        pl.BlockSpec((1, BN, QW), lambda i, q=q: (q, i, 0)) for q in range(NQ)
    ]
    return pl.pallas_call(
        _dense_body,
        grid=(GRID,),
        in_specs=qspecs + [
            pl.BlockSpec((BN, 1), lambda i: (i, 0)),
            pl.BlockSpec((BN, 1), lambda i: (i, 0)),
            pl.BlockSpec((F, H), lambda i: (0, 0)),
            pl.BlockSpec((1, H), lambda i: (0, 0)),
            pl.BlockSpec((H, 1), lambda i: (0, 0)),
        ],
        out_specs=pl.BlockSpec((BN, 1), lambda i: (i, 0)),
        out_shape=jax.ShapeDtypeStruct((N, 1), jnp.float32),
    )(agg, agg, agg, agg, ndst, nsrc, W1, b1, W2)


@functools.partial(
    pl.kernel,
    out_type=jax.ShapeDtypeStruct((2, NPAD), jnp.float32),
    mesh=_mesh,
    scratch_types=[
        pltpu.VMEM((NCHUNK, CW), jnp.int32),
        pltpu.VMEM((NCHUNK, CW), jnp.int32),
        pltpu.VMEM((2, KSUP, CW), jnp.float32),
        pltpu.VMEM((RPT,), jnp.float32),
        pltpu.VMEM((RPT,), jnp.float32),
        pltpu.VMEM((RPT,), jnp.float32),
        pltpu.VMEM((16,), jnp.float32),
        pltpu.VMEM_SHARED((NPAD,), jnp.float32),
        pltpu.SemaphoreType.DMA,
    ],
)
def _layer2(g1, s3, d3, ndp, b2h, o2, src_v, dst_v, gst,
            lbuf, nbuf, obuf, b2v, acc, gsem):
    c = lax.axis_index("c")
    s = lax.axis_index("s")
    pltpu.sync_copy(s3.at[s], src_v)
    pltpu.sync_copy(d3.at[s], dst_v)
    pltpu.sync_copy(b2h, b2v)
    _zero_vec(obuf, RPT)
    pltpu.sync_copy(obuf, acc.at[pl.ds(s * RPT, RPT)])
    plsc.subcore_barrier()

    def fire(t, buf):
        base = t * KSUP
        for b in range(KSUP):
            pltpu.async_copy(g1.at[src_v.at[base + b]], gst.at[buf, b], gsem)

    def drain(buf):
        for b in range(KSUP):
            pltpu.make_async_copy(
                g1.at[pl.ds(0, CW)], gst.at[buf, b], gsem).wait()

    fire(0, 0)

    def body(t, carry):
        pp = lax.rem(t, 2)
        drain(pp)

        @pl.when(t + 1 < NSUP)
        def _():
            fire(t + 1, 1 - pp)
        base = t * KSUP
        for b in range(KSUP):
            pltpu.sync_copy(
                gst.at[pp, b], acc.at[dst_v.at[base + b]], add=True)
        return carry
    lax.fori_loop(0, NSUP, body, 0)
    plsc.subcore_barrier()

    pltpu.sync_copy(acc.at[pl.ds(s * RPT, RPT)], lbuf)
    pltpu.sync_copy(ndp.at[pl.ds(s * RPT, RPT)], nbuf)
    b2r = b2v[...]

    def scale(k, carry):
        sl = pl.ds(k * 16, 16)
        obuf[sl] = lbuf[sl] * nbuf[sl] + b2r
        return carry
    lax.fori_loop(0, RPT // 16, scale, 0)
    pltpu.sync_copy(obuf, o2.at[c, pl.ds(s * RPT, RPT)])


def kernel(features, edge_index, W1, b1, W2, b2):
    src = edge_index[0].astype(jnp.int32)
    dst = edge_index[1].astype(jnp.int32)
    src3 = src.reshape(NSUB, NCHUNK, CW)
    dst3 = dst.reshape(NSUB, NCHUNK, CW)
    e4 = jnp.stack([src3, dst3])
    s8 = jnp.stack([src3 + q * N for q in range(NQ)])

    deg2 = _degrees(e4)
    deg_out = deg2[0, :N, None]
    deg_in = deg2[1, :N, None]

    ycat, nsrc, ndst = _normalize(features, deg_out, deg_in)
    yflat = ycat.reshape(NQ * N, QW)

    agg = _aggregate(yflat, s8, dst3)
    g = _dense(agg, ndst, nsrc, W1, b1.reshape(1, H), W2)

    ndp = jnp.concatenate([ndst[:, 0], jnp.zeros((NPAD - N,), jnp.float32)])
    b2h = jnp.broadcast_to(b2, (16,))
    o2 = _layer2(g[:, 0], src3, dst3, ndp, b2h)
    return o2[0, :N].reshape(N, 1)


# speedup vs baseline: 9.2326x; 1.0060x over previous
import functools

import jax
import jax.numpy as jnp
from jax import lax
from jax.experimental import pallas as pl
from jax.experimental.pallas import tpu as pltpu
from jax.experimental.pallas import tpu_sc as plsc

N = 10000
E = 160000
F = 256
H = 512

NSUB = 16
NCHUNK = 125
CW = 80
NPAD = 10240
RPT = NPAD // NSUB
BN = 400
GRID = N // BN

_mesh = plsc.VectorSubcoreMesh(core_axis_name="c", subcore_axis_name="s")


def _zero_vec(ref, nwords):
    def body(k, carry):
        ref[pl.ds(k * 16, 16)] = jnp.zeros((16,), jnp.float32)
        return carry
    lax.fori_loop(0, nwords // 16, body, 0)


@functools.partial(
    pl.kernel,
    out_type=jax.ShapeDtypeStruct((2, NPAD), jnp.float32),
    mesh=_mesh,
    scratch_types=[
        pltpu.VMEM((NCHUNK, CW), jnp.int32),
        pltpu.VMEM((CW,), jnp.float32),
        pltpu.VMEM((RPT,), jnp.float32),
        pltpu.VMEM_SHARED((NPAD,), jnp.float32),
    ],
)
def _degrees(src3, dst3, deg2, idx_v, ones_v, buf, acc):
    c = lax.axis_index("c")
    s = lax.axis_index("s")

    @pl.when(c == 0)
    def _():
        pltpu.sync_copy(src3.at[s], idx_v)

    @pl.when(c == 1)
    def _():
        pltpu.sync_copy(dst3.at[s], idx_v)
    for l in range(CW // 16):
        ones_v[pl.ds(l * 16, 16)] = jnp.ones((16,), jnp.float32)
    _zero_vec(buf, RPT)
    pltpu.sync_copy(buf, acc.at[pl.ds(s * RPT, RPT)])
    plsc.subcore_barrier()

    def body(j, carry):
        pltpu.sync_copy(ones_v, acc.at[idx_v.at[j]], add=True)
        return carry
    lax.fori_loop(0, NCHUNK, body, 0)
    plsc.subcore_barrier()
    pltpu.sync_copy(acc.at[pl.ds(s * RPT, RPT)], buf)
    pltpu.sync_copy(buf, deg2.at[c, pl.ds(s * RPT, RPT)])


QW = 64
NQ = F // QW


def _normalize_body(feat, dego, degi, y, nsrc, ndst):
    ns = lax.rsqrt(jnp.maximum(dego[...], 1.0))
    nd = lax.rsqrt(jnp.maximum(degi[...], 1.0))
    ysc = feat[...] * ns
    y[0] = ysc[:, :QW]
    y[1] = ysc[:, QW:]
    nsrc[...] = ns
    ndst[...] = nd


def _normalize(features, deg_out, deg_in):
    return pl.pallas_call(
        _normalize_body,
        grid=(GRID, 2),
        in_specs=[
            pl.BlockSpec((BN, 128), lambda i, h: (i, h)),
            pl.BlockSpec((BN, 1), lambda i, h: (i, 0)),
            pl.BlockSpec((BN, 1), lambda i, h: (i, 0)),
        ],
        out_specs=[
            pl.BlockSpec((2, BN, QW), lambda i, h: (h, i, 0)),
            pl.BlockSpec((BN, 1), lambda i, h: (i, 0)),
            pl.BlockSpec((BN, 1), lambda i, h: (i, 0)),
        ],
        out_shape=[
            jax.ShapeDtypeStruct((NQ, N, QW), jnp.float32),
            jax.ShapeDtypeStruct((N, 1), jnp.float32),
            jax.ShapeDtypeStruct((N, 1), jnp.float32),
        ],
    )(features, deg_out, deg_in)


KSUP = 5

NSUP = NCHUNK // KSUP

@functools.partial(
    pl.kernel,
    out_type=jax.ShapeDtypeStruct((NQ, NPAD, QW), jnp.float32),
    mesh=_mesh,
    scratch_types=[
        pltpu.VMEM((NCHUNK, CW), jnp.int32),
        pltpu.VMEM((NCHUNK, CW), jnp.int32),
        pltpu.VMEM((2, KSUP, CW, QW), jnp.float32),
        pltpu.VMEM((16, QW), jnp.float32),
        pltpu.VMEM_SHARED((NPAD, QW), jnp.float32),
        pltpu.SemaphoreType.DMA,
        pltpu.SemaphoreType.DMA,
    ],
    compiler_params=pltpu.CompilerParams(use_tc_tiling_on_sc=False),
)
def _aggregate(yc, s3, d3, agg, src_v, dst_v, st, zb, acc, gsem, ssem):
    c = lax.axis_index("c")
    s = lax.axis_index("s")
    pltpu.sync_copy(d3.at[s], dst_v)
    pltpu.sync_copy(s3.at[s], src_v)

    def zrow(r, carry):
        for l in range(QW // 16):
            zb[r, pl.ds(l * 16, 16)] = jnp.zeros((16,), jnp.float32)
        return carry
    lax.fori_loop(0, 16, zrow, 0)

    def fire(t, buf, tab):
        base = t * KSUP
        for b in range(KSUP):
            pltpu.async_copy(
                tab.at[src_v.at[base + b]], st.at[buf, b], gsem)

    def drain(buf):
        for b in range(KSUP):
            pltpu.make_async_copy(
                yc.at[pl.ds(0, CW)], st.at[buf, b], gsem).wait()

    def fire_sc(t, buf):
        base = t * KSUP
        for b in range(KSUP):
            pltpu.async_copy(
                st.at[buf, b], acc.at[dst_v.at[base + b]], ssem, add=True)

    def drain_sc(buf):
        for b in range(KSUP):
            pltpu.make_async_copy(
                yc.at[pl.ds(0, CW)], st.at[buf, b], ssem).wait()

    for p in range(2):
        q = 2 * c + p
        tab = yc.at[pl.ds(q * N, N)]

        def zcp(t, carry):
            pltpu.sync_copy(zb, acc.at[pl.ds(s * RPT + t * 16, 16)])
            return carry
        lax.fori_loop(0, RPT // 16, zcp, 0)
        plsc.subcore_barrier()

        fire(0, 0, tab)

        def super_step(t, carry):
            pp = lax.rem(t, 2)
            drain(pp)

            @pl.when(t >= 1)
            def _():
                drain_sc(1 - pp)

            @pl.when(t + 1 < NSUP)
            def _():
                fire(t + 1, 1 - pp, tab)
            fire_sc(t, pp)
            return carry
        lax.fori_loop(0, NSUP, super_step, 0)
        drain_sc(lax.rem(NSUP - 1, 2))
        plsc.subcore_barrier()

        def out_cp(t, carry):
            pltpu.sync_copy(acc.at[pl.ds(s * RPT + t * 16, 16)], zb)
            pltpu.sync_copy(zb, agg.at[q, pl.ds(s * RPT + t * 16, 16)])
            return carry
        lax.fori_loop(0, RPT // 16, out_cp, 0)
        if p == 0:
            lax.fori_loop(0, 16, zrow, 0)
            plsc.subcore_barrier()


def _dense_body(a0, a1, a2, a3, nd, ns, w1, b1, w2, g):
    a = jnp.concatenate([a0[0], a1[0], a2[0], a3[0]], axis=1)
    h = lax.dot_general(
        a, w1[...], (((1,), (0,)), ((), ())),
        precision=lax.Precision.HIGHEST, preferred_element_type=jnp.float32)
    h = h * nd[...] + b1[...]
    h = jnp.maximum(h, 0.0) * ns[...]
    g[...] = lax.dot_general(
        h, w2[...], (((1,), (0,)), ((), ())),
        precision=lax.Precision.HIGHEST, preferred_element_type=jnp.float32)


def _dense(agg, ndst, nsrc, W1, b1, W2):
    qspecs = [
        pl.BlockSpec((1, BN, QW), lambda i, q=q: (q, i, 0)) for q in range(NQ)
    ]
    return pl.pallas_call(
        _dense_body,
        grid=(GRID,),
        in_specs=qspecs + [
            pl.BlockSpec((BN, 1), lambda i: (i, 0)),
            pl.BlockSpec((BN, 1), lambda i: (i, 0)),
            pl.BlockSpec((F, H), lambda i: (0, 0)),
            pl.BlockSpec((1, H), lambda i: (0, 0)),
            pl.BlockSpec((H, 1), lambda i: (0, 0)),
        ],
        out_specs=pl.BlockSpec((BN, 1), lambda i: (i, 0)),
        out_shape=jax.ShapeDtypeStruct((N, 1), jnp.float32),
    )(agg, agg, agg, agg, ndst, nsrc, W1, b1, W2)


@functools.partial(
    pl.kernel,
    out_type=jax.ShapeDtypeStruct((2, NPAD), jnp.float32),
    mesh=_mesh,
    scratch_types=[
        pltpu.VMEM((NCHUNK, CW), jnp.int32),
        pltpu.VMEM((NCHUNK, CW), jnp.int32),
        pltpu.VMEM((2, KSUP, CW), jnp.float32),
        pltpu.VMEM((RPT,), jnp.float32),
        pltpu.VMEM((RPT,), jnp.float32),
        pltpu.VMEM((RPT,), jnp.float32),
        pltpu.VMEM((16,), jnp.float32),
        pltpu.VMEM_SHARED((NPAD,), jnp.float32),
        pltpu.SemaphoreType.DMA,
        pltpu.SemaphoreType.DMA,
    ],
)
def _layer2(g1, s3, d3, ndp, b2h, o2, src_v, dst_v, gst,
            lbuf, nbuf, obuf, b2v, acc, gsem, ssem):
    c = lax.axis_index("c")
    s = lax.axis_index("s")
    pltpu.sync_copy(s3.at[s], src_v)
    pltpu.sync_copy(d3.at[s], dst_v)
    pltpu.sync_copy(b2h, b2v)
    _zero_vec(obuf, RPT)
    pltpu.sync_copy(obuf, acc.at[pl.ds(s * RPT, RPT)])
    plsc.subcore_barrier()

    def fire(t, buf):
        base = t * KSUP
        for b in range(KSUP):
            pltpu.async_copy(g1.at[src_v.at[base + b]], gst.at[buf, b], gsem)

    def drain(buf, sem):
        for b in range(KSUP):
            pltpu.make_async_copy(
                g1.at[pl.ds(0, CW)], gst.at[buf, b], sem).wait()

    def fire_sc(t, buf):
        base = t * KSUP
        for b in range(KSUP):
            pltpu.async_copy(
                gst.at[buf, b], acc.at[dst_v.at[base + b]], ssem, add=True)

    fire(0, 0)

    def body(t, carry):
        pp = lax.rem(t, 2)
        drain(pp, gsem)

        @pl.when(t >= 1)
        def _():
            drain(1 - pp, ssem)

        @pl.when(t + 1 < NSUP)
        def _():
            fire(t + 1, 1 - pp)
        fire_sc(t, pp)
        return carry
    lax.fori_loop(0, NSUP, body, 0)
    drain(lax.rem(NSUP - 1, 2), ssem)
    plsc.subcore_barrier()

    pltpu.sync_copy(acc.at[pl.ds(s * RPT, RPT)], lbuf)
    pltpu.sync_copy(ndp.at[pl.ds(s * RPT, RPT)], nbuf)
    b2r = b2v[...]

    def scale(k, carry):
        sl = pl.ds(k * 16, 16)
        obuf[sl] = lbuf[sl] * nbuf[sl] + b2r
        return carry
    lax.fori_loop(0, RPT // 16, scale, 0)
    pltpu.sync_copy(obuf, o2.at[c, pl.ds(s * RPT, RPT)])


def kernel(features, edge_index, W1, b1, W2, b2):
    src = edge_index[0].astype(jnp.int32)
    dst = edge_index[1].astype(jnp.int32)
    src3 = src.reshape(NSUB, NCHUNK, CW)
    dst3 = dst.reshape(NSUB, NCHUNK, CW)

    deg2 = _degrees(src3, dst3)
    deg_out = deg2[0, :N, None]
    deg_in = deg2[1, :N, None]

    ycat, nsrc, ndst = _normalize(features, deg_out, deg_in)
    yflat = ycat.reshape(NQ * N, QW)

    agg = _aggregate(yflat, src3, dst3)
    g = _dense(agg, ndst, nsrc, W1, b1.reshape(1, H), W2)

    ndp = jnp.concatenate([ndst[:, 0], jnp.zeros((NPAD - N,), jnp.float32)])
    b2h = jnp.broadcast_to(b2, (16,))
    o2 = _layer2(g[:, 0], src3, dst3, ndp, b2h)
    return o2[0, :N].reshape(N, 1)


# speedup vs baseline: 9.4165x; 1.0199x over previous
import functools

import jax
import jax.numpy as jnp
from jax import lax
from jax.experimental import pallas as pl
from jax.experimental.pallas import tpu as pltpu
from jax.experimental.pallas import tpu_sc as plsc

N = 10000
E = 160000
F = 256
H = 512

NSUB = 16
NCHUNK = 125
CW = 80
NPAD = 10240
RPT = NPAD // NSUB
BN = 400
GRID = N // BN

_mesh = plsc.VectorSubcoreMesh(core_axis_name="c", subcore_axis_name="s")


def _zero_vec(ref, nwords):
    def body(k, carry):
        ref[pl.ds(k * 16, 16)] = jnp.zeros((16,), jnp.float32)
        return carry
    lax.fori_loop(0, nwords // 16, body, 0)


@functools.partial(
    pl.kernel,
    out_type=jax.ShapeDtypeStruct((2, NPAD), jnp.float32),
    mesh=_mesh,
    scratch_types=[
        pltpu.VMEM((NCHUNK, CW), jnp.int32),
        pltpu.VMEM((CW,), jnp.float32),
        pltpu.VMEM((RPT,), jnp.float32),
        pltpu.VMEM_SHARED((NPAD,), jnp.float32),
    ],
)
def _degrees(src3, dst3, deg2, idx_v, ones_v, buf, acc):
    c = lax.axis_index("c")
    s = lax.axis_index("s")

    @pl.when(c == 0)
    def _():
        pltpu.sync_copy(src3.at[s], idx_v)

    @pl.when(c == 1)
    def _():
        pltpu.sync_copy(dst3.at[s], idx_v)
    for l in range(CW // 16):
        ones_v[pl.ds(l * 16, 16)] = jnp.ones((16,), jnp.float32)
    _zero_vec(buf, RPT)
    pltpu.sync_copy(buf, acc.at[pl.ds(s * RPT, RPT)])
    plsc.subcore_barrier()

    def body(j, carry):
        pltpu.sync_copy(ones_v, acc.at[idx_v.at[j]], add=True)
        return carry
    lax.fori_loop(0, NCHUNK, body, 0)
    plsc.subcore_barrier()
    pltpu.sync_copy(acc.at[pl.ds(s * RPT, RPT)], buf)
    pltpu.sync_copy(buf, deg2.at[c, pl.ds(s * RPT, RPT)])


QW = 64
NQ = F // QW


def _normalize_body(feat, dego, degi, y, nsrc, ndst):
    ns = lax.rsqrt(jnp.maximum(dego[...], 1.0))
    nd = lax.rsqrt(jnp.maximum(degi[...], 1.0))
    ysc = feat[...] * ns
    y[0] = ysc[:, :QW]
    y[1] = ysc[:, QW:]
    nsrc[...] = ns
    ndst[...] = nd


def _normalize(features, deg_out, deg_in):
    return pl.pallas_call(
        _normalize_body,
        grid=(GRID, 2),
        in_specs=[
            pl.BlockSpec((BN, 128), lambda i, h: (i, h)),
            pl.BlockSpec((BN, 1), lambda i, h: (i, 0)),
            pl.BlockSpec((BN, 1), lambda i, h: (i, 0)),
        ],
        out_specs=[
            pl.BlockSpec((2, BN, QW), lambda i, h: (h, i, 0)),
            pl.BlockSpec((BN, 1), lambda i, h: (i, 0)),
            pl.BlockSpec((BN, 1), lambda i, h: (i, 0)),
        ],
        out_shape=[
            jax.ShapeDtypeStruct((NQ, N, QW), jnp.float32),
            jax.ShapeDtypeStruct((N, 1), jnp.float32),
            jax.ShapeDtypeStruct((N, 1), jnp.float32),
        ],
    )(features, deg_out, deg_in)


KSUP = 5

NSUP = NCHUNK // KSUP

@functools.partial(
    pl.kernel,
    out_type=jax.ShapeDtypeStruct((NPAD, F), jnp.float32),
    mesh=_mesh,
    scratch_types=[
        pltpu.VMEM((NCHUNK, CW), jnp.int32),
        pltpu.VMEM((NCHUNK, CW), jnp.int32),
        pltpu.VMEM((2, KSUP, CW, QW), jnp.float32),
        pltpu.VMEM((16, QW), jnp.float32),
        pltpu.VMEM_SHARED((NPAD, QW), jnp.float32),
        pltpu.SemaphoreType.DMA,
        pltpu.SemaphoreType.DMA,
    ],
    compiler_params=pltpu.CompilerParams(use_tc_tiling_on_sc=False),
)
def _aggregate(yc, s3, d3, agg, src_v, dst_v, st, zb, acc, gsem, ssem):
    c = lax.axis_index("c")
    s = lax.axis_index("s")
    pltpu.sync_copy(d3.at[s], dst_v)
    pltpu.sync_copy(s3.at[s], src_v)

    def zrow(r, carry):
        for l in range(QW // 16):
            zb[r, pl.ds(l * 16, 16)] = jnp.zeros((16,), jnp.float32)
        return carry
    lax.fori_loop(0, 16, zrow, 0)

    def fire(t, buf, tab):
        base = t * KSUP
        for b in range(KSUP):
            pltpu.async_copy(
                tab.at[src_v.at[base + b]], st.at[buf, b], gsem)

    def drain(buf):
        for b in range(KSUP):
            pltpu.make_async_copy(
                yc.at[0, pl.ds(0, CW)], st.at[buf, b], gsem).wait()

    def fire_sc(t, buf):
        base = t * KSUP
        for b in range(KSUP):
            pltpu.async_copy(
                st.at[buf, b], acc.at[dst_v.at[base + b]], ssem, add=True)

    def drain_sc(buf):
        for b in range(KSUP):
            pltpu.make_async_copy(
                yc.at[0, pl.ds(0, CW)], st.at[buf, b], ssem).wait()

    for p in range(2):
        q = 2 * c + p
        tab = yc.at[q]

        def zcp(t, carry):
            pltpu.sync_copy(zb, acc.at[pl.ds(s * RPT + t * 16, 16)])
            return carry
        lax.fori_loop(0, RPT // 16, zcp, 0)
        plsc.subcore_barrier()

        fire(0, 0, tab)

        def super_step(t, carry):
            pp = lax.rem(t, 2)
            drain(pp)

            @pl.when(t >= 1)
            def _():
                drain_sc(1 - pp)

            @pl.when(t + 1 < NSUP)
            def _():
                fire(t + 1, 1 - pp, tab)
            fire_sc(t, pp)
            return carry
        lax.fori_loop(0, NSUP, super_step, 0)
        drain_sc(lax.rem(NSUP - 1, 2))
        plsc.subcore_barrier()

        def out_cp(t, carry):
            pltpu.sync_copy(acc.at[pl.ds(s * RPT + t * 16, 16)], zb)
            pltpu.sync_copy(
                zb, agg.at[pl.ds(s * RPT + t * 16, 16), pl.ds(q * QW, QW)])
            return carry
        lax.fori_loop(0, RPT // 16, out_cp, 0)
        if p == 0:
            lax.fori_loop(0, 16, zrow, 0)
            plsc.subcore_barrier()


def _dense_body(a, nd, ns, w1, b1, w2, g):
    h = lax.dot_general(
        a[...], w1[...], (((1,), (0,)), ((), ())),
        precision=lax.Precision.HIGHEST, preferred_element_type=jnp.float32)
    h = h * nd[...] + b1[...]
    h = jnp.maximum(h, 0.0) * ns[...]
    g[...] = lax.dot_general(
        h, w2[...], (((1,), (0,)), ((), ())),
        precision=lax.Precision.HIGHEST, preferred_element_type=jnp.float32)


def _dense(agg, ndst, nsrc, W1, b1, W2):
    return pl.pallas_call(
        _dense_body,
        grid=(GRID,),
        in_specs=[
            pl.BlockSpec((BN, F), lambda i: (i, 0)),
            pl.BlockSpec((BN, 1), lambda i: (i, 0)),
            pl.BlockSpec((BN, 1), lambda i: (i, 0)),
            pl.BlockSpec((F, H), lambda i: (0, 0)),
            pl.BlockSpec((1, H), lambda i: (0, 0)),
            pl.BlockSpec((H, 1), lambda i: (0, 0)),
        ],
        out_specs=pl.BlockSpec((BN, 1), lambda i: (i, 0)),
        out_shape=jax.ShapeDtypeStruct((N, 1), jnp.float32),
    )(agg, ndst, nsrc, W1, b1, W2)


@functools.partial(
    pl.kernel,
    out_type=jax.ShapeDtypeStruct((2, NPAD), jnp.float32),
    mesh=_mesh,
    scratch_types=[
        pltpu.VMEM((NCHUNK, CW), jnp.int32),
        pltpu.VMEM((NCHUNK, CW), jnp.int32),
        pltpu.VMEM((2, KSUP, CW), jnp.float32),
        pltpu.VMEM((RPT,), jnp.float32),
        pltpu.VMEM((RPT,), jnp.float32),
        pltpu.VMEM((RPT,), jnp.float32),
        pltpu.VMEM((16,), jnp.float32),
        pltpu.VMEM_SHARED((NPAD,), jnp.float32),
        pltpu.SemaphoreType.DMA,
        pltpu.SemaphoreType.DMA,
    ],
)
def _layer2(g1, s3, d3, ndp, b2h, o2, src_v, dst_v, gst,
            lbuf, nbuf, obuf, b2v, acc, gsem, ssem):
    c = lax.axis_index("c")
    s = lax.axis_index("s")
    pltpu.sync_copy(s3.at[s], src_v)
    pltpu.sync_copy(d3.at[s], dst_v)
    pltpu.sync_copy(b2h, b2v)
    _zero_vec(obuf, RPT)
    pltpu.sync_copy(obuf, acc.at[pl.ds(s * RPT, RPT)])
    plsc.subcore_barrier()

    def fire(t, buf):
        base = t * KSUP
        for b in range(KSUP):
            pltpu.async_copy(g1.at[src_v.at[base + b]], gst.at[buf, b], gsem)

    def drain(buf, sem):
        for b in range(KSUP):
            pltpu.make_async_copy(
                g1.at[pl.ds(0, CW)], gst.at[buf, b], sem).wait()

    def fire_sc(t, buf):
        base = t * KSUP
        for b in range(KSUP):
            pltpu.async_copy(
                gst.at[buf, b], acc.at[dst_v.at[base + b]], ssem, add=True)

    fire(0, 0)

    def body(t, carry):
        pp = lax.rem(t, 2)
        drain(pp, gsem)

        @pl.when(t >= 1)
        def _():
            drain(1 - pp, ssem)

        @pl.when(t + 1 < NSUP)
        def _():
            fire(t + 1, 1 - pp)
        fire_sc(t, pp)
        return carry
    lax.fori_loop(0, NSUP, body, 0)
    drain(lax.rem(NSUP - 1, 2), ssem)
    plsc.subcore_barrier()

    pltpu.sync_copy(acc.at[pl.ds(s * RPT, RPT)], lbuf)
    pltpu.sync_copy(ndp.at[pl.ds(s * RPT, RPT)], nbuf)
    b2r = b2v[...]

    def scale(k, carry):
        sl = pl.ds(k * 16, 16)
        obuf[sl] = lbuf[sl] * nbuf[sl] + b2r
        return carry
    lax.fori_loop(0, RPT // 16, scale, 0)
    pltpu.sync_copy(obuf, o2.at[c, pl.ds(s * RPT, RPT)])


def kernel(features, edge_index, W1, b1, W2, b2):
    src = edge_index[0].astype(jnp.int32)
    dst = edge_index[1].astype(jnp.int32)
    src3 = src.reshape(NSUB, NCHUNK, CW)
    dst3 = dst.reshape(NSUB, NCHUNK, CW)

    deg2 = _degrees(src3, dst3)
    deg_out = deg2[0, :N, None]
    deg_in = deg2[1, :N, None]

    ycat, nsrc, ndst = _normalize(features, deg_out, deg_in)

    agg = _aggregate(ycat, src3, dst3)
    g = _dense(agg, ndst, nsrc, W1, b1.reshape(1, H), W2)

    ndp = jnp.concatenate([ndst[:, 0], jnp.zeros((NPAD - N,), jnp.float32)])
    b2h = jnp.broadcast_to(b2, (16,))
    o2 = _layer2(g[:, 0], src3, dst3, ndp, b2h)
    return o2[0, :N].reshape(N, 1)
